# baseline (speedup 1.0000x reference)
import functools

import jax
import jax.numpy as jnp
import numpy as np
from jax.experimental import pallas as pl
from jax.experimental.pallas import tpu as pltpu

S = 96
KW = float(np.sqrt(3.0 * 3.0 / np.pi))
EPS = 1e-8
D = 192
YP = D + 2
LZ = 256
R = D * YP
DUMP = R
R2 = R + 2
B = 50
BS = B * S


def _proj_body(img_hbm, idx_ref, wts_ref, p_ref, img_vmem, idx_smem,
               wts_smem, sems):
    b = pl.program_id(1)

    @pl.when(b == 0)
    def _():
        cp = pltpu.make_async_copy(img_hbm, img_vmem, sems.at[0])
        cp.start()
        cp.wait()

    ci = pltpu.make_async_copy(idx_ref.at[0], idx_smem, sems.at[1])
    ci.start()
    cw = pltpu.make_async_copy(wts_ref.at[0], wts_smem, sems.at[2])
    cw.start()
    ci.wait()
    cw.wait()

    io = jax.lax.broadcasted_iota(jnp.int32, (1, LZ), 1)
    iol = jax.lax.broadcasted_iota(jnp.int32, (1, B), 1)
    zero = jnp.zeros((1, LZ), jnp.float32)

    def lor_body(l, pacc):
        def grp_body(g, acc):
            s0 = l * S + g * 8
            for mi in range(8):
                s = s0 + mi
                r0 = idx_smem[0, s]
                r1 = idx_smem[1, s]
                iz = idx_smem[2, s]
                fz = wts_smem[0, s]
                mx0 = wts_smem[1, s]
                mx1 = wts_smem[2, s]
                fy = wts_smem[3, s]
                g00 = img_vmem[r0]
                g01 = img_vmem[r0 + 1]
                g10 = img_vmem[r1]
                g11 = img_vmem[r1 + 1]
                my0 = 1.0 - fy
                gxy = (g00 * (mx0 * my0) + g01 * (mx0 * fy) +
                       g10 * (mx1 * my0) + g11 * (mx1 * fy))
                zv = (jnp.where(io == iz, 1.0 - fz, 0.0) +
                      jnp.where(io == iz + 1, fz, 0.0))
                acc = acc + gxy * zv
            return acc

        acc = jax.lax.fori_loop(0, S // 8, grp_body, zero)
        tot = jnp.sum(acc, axis=1, keepdims=True)
        return pacc + jnp.where(iol == l, tot, 0.0)

    pacc = jax.lax.fori_loop(0, B, lor_body, jnp.zeros((1, B), jnp.float32))
    p_ref[0] = pacc


def _bp_body(nb2, idx_ref, wts_ref, c_ref, out_hbm, acc_vmem, idx_smem,
             wts_smem, c_smem, sems):
    core = pl.program_id(0)
    b = pl.program_id(1)

    @pl.when(b == 0)
    def _():
        def zb(i, carry):
            acc_vmem[pl.ds(i * 50, 50)] = jnp.zeros((50, 1, LZ), jnp.float32)
            return carry

        jax.lax.fori_loop(0, R2 // 50, zb, 0)

    ci = pltpu.make_async_copy(idx_ref.at[0], idx_smem, sems.at[0])
    ci.start()
    cw = pltpu.make_async_copy(wts_ref.at[0], wts_smem, sems.at[1])
    cw.start()
    cc = pltpu.make_async_copy(c_ref.at[0, 0], c_smem, sems.at[2])
    cc.start()
    ci.wait()
    cw.wait()
    cc.wait()

    io = jax.lax.broadcasted_iota(jnp.int32, (1, LZ), 1)

    def lor_body(l, carry):
        cl = c_smem[l]

        def grp_body(g, carry2):
            s0 = l * S + g * 8
            for mi in range(8):
                s = s0 + mi
                r0 = idx_smem[0, s]
                r1 = idx_smem[1, s]
                iz = idx_smem[2, s]
                fz = wts_smem[0, s]
                mx0 = wts_smem[1, s]
                mx1 = wts_smem[2, s]
                fy = wts_smem[3, s]
                my0 = 1.0 - fy
                zvc = (jnp.where(io == iz, 1.0 - fz, 0.0) +
                       jnp.where(io == iz + 1, fz, 0.0)) * cl
                v00 = acc_vmem[r0] + zvc * (mx0 * my0)
                v01 = acc_vmem[r0 + 1] + zvc * (mx0 * fy)
                v10 = acc_vmem[r1] + zvc * (mx1 * my0)
                v11 = acc_vmem[r1 + 1] + zvc * (mx1 * fy)
                acc_vmem[r0] = v00
                acc_vmem[r0 + 1] = v01
                acc_vmem[r1] = v10
                acc_vmem[r1 + 1] = v11
            return carry2

        jax.lax.fori_loop(0, S // 8, grp_body, 0)
        return carry

    jax.lax.fori_loop(0, B, lor_body, 0)

    @pl.when(b == nb2 - 1)
    def _():
        co = pltpu.make_async_copy(acc_vmem, out_hbm.at[pl.ds(core * R2, R2)],
                                   sems.at[3])
        co.start()
        co.wait()


def _combine_body(img, eff, z0, z1, x0, x1, y0, y1, o):
    o[...] = img[...] / (eff[...] + EPS) * (
        z0[...] + z1[...] + x0[...] + x1[...] + y0[...] + y1[...])


def _make_proj(nbtot):
    nb2 = nbtot // 2
    return pl.pallas_call(
        _proj_body,
        grid=(2, nb2),
        in_specs=[
            pl.BlockSpec(memory_space=pl.ANY),
            pl.BlockSpec((1, 3, BS), lambda c, b: (c * nb2 + b, 0, 0)),
            pl.BlockSpec((1, 4, BS), lambda c, b: (c * nb2 + b, 0, 0)),
        ],
        out_specs=pl.BlockSpec((1, 1, B), lambda c, b: (c * nb2 + b, 0, 0)),
        out_shape=jax.ShapeDtypeStruct((nbtot, 1, B), jnp.float32),
        scratch_shapes=[
            pltpu.VMEM((R2, 1, LZ), jnp.float32),
            pltpu.SMEM((3, BS), jnp.int32),
            pltpu.SMEM((4, BS), jnp.float32),
            pltpu.SemaphoreType.DMA((3,)),
        ],
        compiler_params=pltpu.CompilerParams(
            dimension_semantics=("parallel", "arbitrary"),
            vmem_limit_bytes=52 * 1024 * 1024,
        ),
        name="recon_proj",
    )


def _make_bp(nbtot):
    nb2 = nbtot // 2
    return pl.pallas_call(
        functools.partial(_bp_body, nb2),
        grid=(2, nb2),
        in_specs=[
            pl.BlockSpec((1, 3, BS), lambda c, b: (c * nb2 + b, 0, 0)),
            pl.BlockSpec((1, 4, BS), lambda c, b: (c * nb2 + b, 0, 0)),
            pl.BlockSpec((1, 1, B), lambda c, b: (c * nb2 + b, 0, 0)),
        ],
        out_specs=pl.BlockSpec(memory_space=pl.ANY),
        out_shape=jax.ShapeDtypeStruct((2 * R2, 1, LZ), jnp.float32),
        scratch_shapes=[
            pltpu.VMEM((R2, 1, LZ), jnp.float32),
            pltpu.SMEM((3, BS), jnp.int32),
            pltpu.SMEM((4, BS), jnp.float32),
            pltpu.SMEM((B,), jnp.float32),
            pltpu.SemaphoreType.DMA((4,)),
        ],
        compiler_params=pltpu.CompilerParams(
            dimension_semantics=("parallel", "arbitrary"),
            vmem_limit_bytes=52 * 1024 * 1024,
        ),
        name="recon_bp",
    )


def _make_combine():
    nb = D // 8 // 2

    def im_img(c, b):
        return (c * nb + b, 0, 0)

    def im_h1(c, b):
        return (2 * nb + c * nb + b, 0, 0)

    return pl.pallas_call(
        _combine_body,
        grid=(2, nb),
        in_specs=[
            pl.BlockSpec((8, D, D), im_img),
            pl.BlockSpec((8, D, D), im_img),
            pl.BlockSpec((8, D, D), im_img),
            pl.BlockSpec((8, D, D), im_h1),
            pl.BlockSpec((8, D, D), im_img),
            pl.BlockSpec((8, D, D), im_h1),
            pl.BlockSpec((8, D, D), im_img),
            pl.BlockSpec((8, D, D), im_h1),
        ],
        out_specs=pl.BlockSpec((8, D, D), im_img),
        out_shape=jax.ShapeDtypeStruct((D, D, D), jnp.float32),
        compiler_params=pltpu.CompilerParams(
            dimension_semantics=("parallel", "arbitrary"),
            vmem_limit_bytes=52 * 1024 * 1024,
        ),
        name="recon_combine",
    )


def _vox_and_seg(lors, center, size):
    p1 = lors[:, :3]
    p2 = lors[:, 3:6]
    t = (jnp.arange(S, dtype=p1.dtype) + 0.5) / S
    pts = p1[:, None, :] + t[None, :, None] * (p2 - p1)[:, None, :]
    dims_f = jnp.full((3,), D, dtype=pts.dtype)
    vsz = size / dims_f
    vox = (pts - (center - 0.5 * size)) / vsz - 0.5
    seg = jnp.linalg.norm(p2 - p1, axis=-1) / S
    return vox, seg


def _mk_arrays(vox, ax, ay, az, nbtot):
    vx = vox[..., ax]
    vy = vox[..., ay]
    vz = vox[..., az]
    fxi = jnp.floor(vx)
    fyi = jnp.floor(vy)
    fzi = jnp.floor(vz)
    fx = vx - fxi
    fy = vy - fyi
    fz = vz - fzi
    ix = jnp.clip(fxi.astype(jnp.int32), -1, D)
    iy = jnp.clip(fyi.astype(jnp.int32), -1, D - 1)
    iz = jnp.clip(fzi.astype(jnp.int32), -1, D)
    valid0 = (ix >= 0) & (ix <= D - 1)
    valid1 = ix + 1 <= D - 1
    mx0 = jnp.where(valid0, 1.0 - fx, 0.0)
    mx1 = jnp.where(valid1, fx, 0.0)
    iyp = iy + 1
    r0 = jnp.where(valid0, ix * YP + iyp, DUMP)
    r1 = jnp.where(valid1, (ix + 1) * YP + iyp, DUMP)
    idx = jnp.stack([r0.reshape(nbtot, BS), r1.reshape(nbtot, BS),
                     iz.reshape(nbtot, BS)], axis=1)
    wts = jnp.stack([fz.reshape(nbtot, BS), mx0.reshape(nbtot, BS),
                     mx1.reshape(nbtot, BS), fy.reshape(nbtot, BS)], axis=1)
    return idx, wts


def _prep_img(img, perm):
    ip = jnp.transpose(img, perm) if perm is not None else img
    ip = jnp.pad(ip, ((0, 0), (1, 1), (0, LZ - D)))
    ip = ip.reshape(R, 1, LZ)
    return jnp.pad(ip, ((0, 2), (0, 0), (0, 0)))


def kernel(image, efficiency_map, grid, center, size, xlors, ylors, zlors):
    n = xlors.shape[0]
    nbtot = n // B
    proj = _make_proj(nbtot)
    bp = _make_bp(nbtot)

    parts = []
    dirs = (
        (zlors, None, (0, 1, 2)),
        (xlors, (2, 0, 1), (1, 2, 0)),
        (ylors, (1, 0, 2), (1, 0, 2)),
    )
    for lors, img_perm, bp_perm in dirs:
        vox, seg = _vox_and_seg(lors, center, size)
        pidx, pwts = _mk_arrays(vox, 0, 1, 2, nbtot)
        praw = proj(_prep_img(image, img_perm), pidx, pwts)
        skw = (seg * KW).reshape(nbtot, 1, B)
        cvals = praw * skw * skw
        if bp_perm == (0, 1, 2):
            bidx, bwts = pidx, pwts
        else:
            bidx, bwts = _mk_arrays(vox, *bp_perm, nbtot)
        bpart = bp(bidx, bwts, cvals)
        q = bpart.reshape(2, R2, LZ)[:, :R, :D]
        q = q.reshape(2, D, YP, D)[:, :, 1:D + 1, :]
        parts.append(q.reshape(2 * D, D, D))

    combine = _make_combine()
    return combine(image, efficiency_map, parts[0], parts[0], parts[1],
                   parts[1], parts[2], parts[2])
